# baseline (speedup 1.0000x reference)
import functools

import numpy as np
import jax
import jax.numpy as jnp
from jax import lax
from jax.experimental import pallas as pl
from jax.experimental.pallas import tpu as pltpu
from jax.experimental.pallas import tpu_sc as plsc

_DIM = 64
_MAX_LEN = 256

NC = 2
NS = 16
NW = NC * NS
BLK = 128
ITILE = 8
L = 16


def _sinusoidal_pe(max_len, dim):
    pos = np.arange(max_len, dtype=np.float32)[:, None]
    i = np.arange(0, dim, 2, dtype=np.float32)[None, :]
    angle = pos / np.power(10000.0, i / dim)
    pe = np.zeros((max_len, dim), dtype=np.float32)
    pe[:, 0::2] = np.sin(angle)
    pe[:, 1::2] = np.cos(angle)
    return pe


@functools.partial(jax.jit, static_argnums=(3, 4))
def _run(ids_t, pe, table2, B, S):
    D = _DIM
    P = 2 * D
    NP = S // 2
    ntile = S // ITILE

    mesh = plsc.VectorSubcoreMesh(core_axis_name="c", subcore_axis_name="s")

    @functools.partial(
        pl.kernel,
        mesh=mesh,
        out_type=jax.ShapeDtypeStruct((B, NP, P), jnp.float32),
        scratch_types=[
            pltpu.VMEM((2, ITILE, BLK), jnp.int32),
            pltpu.VMEM((2, 2, BLK), jnp.int32),
            pltpu.VMEM((2, 2, BLK, P), jnp.float32),
            pltpu.VMEM((2, BLK, P), jnp.float32),
            pltpu.VMEM((S, D), jnp.float32),
            pltpu.SemaphoreType.DMA,
            pltpu.SemaphoreType.DMA,
            pltpu.SemaphoreType.DMA,
        ],
    )
    def body(ids_hbm, pe_hbm, table_hbm, out_hbm,
             idx_v, pidx_v, raw_v, sto_v, pe_v, isem, gsem, ssem):
        wid = lax.axis_index("s") * NC + lax.axis_index("c")
        b0 = wid * BLK

        pltpu.sync_copy(pe_hbm, pe_v)

        def idx_src(t):
            return ids_hbm.at[pl.ds(t * ITILE, ITILE), pl.ds(b0, BLK)]

        def gather_refs(p, b, u):
            return (table_hbm.at[pidx_v.at[b, u]], raw_v.at[b, u])

        def store_refs(p, b):
            return (sto_v.at[b], out_hbm.at[pl.ds(b0, BLK), p, :])

        def shift_and_gather(p, b):
            tb = (p // (ITILE // 2)) % 2
            for u in range(2):
                r = (2 * p + u) % ITILE
                for g in range(BLK // L):
                    sl = pl.ds(g * L, L)
                    pidx_v[b, u, sl] = lax.shift_right_logical(
                        idx_v[tb, r, sl], 1)
            for u in range(2):
                pltpu.async_copy(*gather_refs(p, b, u), gsem)

        def wait_gathers(p, b):
            for u in range(2):
                pltpu.make_async_copy(*gather_refs(p, b, u), gsem).wait()

        def select_pack(p, b):
            tb = (p // (ITILE // 2)) % 2
            for u in range(2):
                r = (2 * p + u) % ITILE
                pr = [pe_v[2 * p + u, pl.ds(16 * j, 16)]
                      for j in range(D // 16)]

                def row_group(g, carry):
                    iv = idx_v[tb, r, pl.ds(g * L, L)]
                    hv = (iv & 1) * D
                    for lane in range(L):
                        row = g * L + lane
                        off = hv[lane]
                        for j in range(D // 16):
                            src = raw_v[b, u, row, pl.ds(off + 16 * j, 16)]
                            sto_v[b, row, pl.ds(u * D + 16 * j, 16)] = (
                                src + pr[j])
                    return carry

                lax.fori_loop(0, BLK // L, row_group, 0)

        def store(p, b):
            pltpu.async_copy(*store_refs(p, b), ssem)

        def wait_store(p, b):
            pltpu.make_async_copy(*store_refs(p, b), ssem).wait()

        pltpu.sync_copy(idx_src(0), idx_v.at[0])
        pltpu.async_copy(idx_src(1), idx_v.at[1], isem)
        shift_and_gather(0, 0)

        ppt = ITILE // 2

        def half(g, b):
            p = 2 * g + b
            nb = b ^ 1

            @pl.when(p >= 1)
            def _():
                wait_store(p - 1, nb)

            @pl.when(jnp.logical_and(p + 1 < NP, (p + 1) % ppt == 0))
            def _():
                pltpu.make_async_copy(idx_src((p + 1) // ppt),
                                      idx_v.at[((p + 1) // ppt) % 2],
                                      isem).wait()

            wait_gathers(p, b)

            @pl.when(p + 1 < NP)
            def _():
                shift_and_gather(p + 1, nb)

            select_pack(p, b)
            store(p, b)

            @pl.when(jnp.logical_and((p + 1) % ppt == 0,
                                     (p + 1) // ppt + 1 < ntile))
            def _():
                pltpu.async_copy(idx_src((p + 1) // ppt + 1),
                                 idx_v.at[(p // ppt) % 2], isem)

        def outer(g, carry):
            half(g, 0)
            half(g, 1)
            return carry

        lax.fori_loop(0, NP // 2, outer, 0)
        wait_store(NP - 1, 1)

    return body(ids_t, pe, table2)


def kernel(input, tok_table):
    B, S = input.shape
    V, D = tok_table.shape
    pe = jnp.asarray(_sinusoidal_pe(_MAX_LEN, D)[:S])
    ids_t = input.T.astype(jnp.int32)
    table2 = tok_table.reshape(V // 2, 2 * D)
    out = _run(ids_t, pe, table2, B, S)
    return out.reshape(B, S, D)


# speedup vs baseline: 1.7878x; 1.7878x over previous
import functools

import numpy as np
import jax
import jax.numpy as jnp
from jax import lax
from jax.experimental import pallas as pl
from jax.experimental.pallas import tpu as pltpu
from jax.experimental.pallas import tpu_sc as plsc

_DIM = 64
_MAX_LEN = 256

NC = 2
NS = 16
NW = NC * NS
BLK = 128
ITILE = 8


def _sinusoidal_pe(max_len, dim):
    pos = np.arange(max_len, dtype=np.float32)[:, None]
    i = np.arange(0, dim, 2, dtype=np.float32)[None, :]
    angle = pos / np.power(10000.0, i / dim)
    pe = np.zeros((max_len, dim), dtype=np.float32)
    pe[:, 0::2] = np.sin(angle)
    pe[:, 1::2] = np.cos(angle)
    return pe


@functools.partial(jax.jit, static_argnums=(3, 4))
def _run(ids_t, pe, table3, B, S):
    D = _DIM
    ntile = S // ITILE

    mesh = plsc.VectorSubcoreMesh(core_axis_name="c", subcore_axis_name="s")

    @functools.partial(
        pl.kernel,
        mesh=mesh,
        out_type=jax.ShapeDtypeStruct((B, S, D), jnp.float32),
        scratch_types=[
            pltpu.VMEM((2, ITILE, BLK), jnp.int32),
            pltpu.VMEM((2, BLK, 1, D), jnp.float32),
            pltpu.VMEM((S, D), jnp.float32),
            pltpu.SemaphoreType.DMA,
            pltpu.SemaphoreType.DMA,
            pltpu.SemaphoreType.DMA,
        ],
    )
    def body(ids_hbm, pe_hbm, table_hbm, out_hbm,
             idx_v, rows_v, pe_v, isem, gsem, ssem):
        wid = lax.axis_index("s") * NC + lax.axis_index("c")
        b0 = wid * BLK

        pltpu.sync_copy(pe_hbm, pe_v)

        def idx_src(t):
            return ids_hbm.at[pl.ds(t * ITILE, ITILE), pl.ds(b0, BLK)]

        def gather_refs(c, b):
            return (table_hbm.at[idx_v.at[(c // ITILE) % 2, c % ITILE]],
                    rows_v.at[b])

        def store_refs(c, b):
            return (rows_v.at[b],
                    out_hbm.at[pl.ds(b0, BLK), pl.ds(c, 1), :])

        def gather(c, b):
            pltpu.async_copy(*gather_refs(c, b), gsem)

        def wait_gather(c, b):
            pltpu.make_async_copy(*gather_refs(c, b), gsem).wait()

        def store(c, b):
            pltpu.async_copy(*store_refs(c, b), ssem)

        def wait_store(c, b):
            pltpu.make_async_copy(*store_refs(c, b), ssem).wait()

        pltpu.sync_copy(idx_src(0), idx_v.at[0])
        pltpu.async_copy(idx_src(1), idx_v.at[1], isem)
        gather(0, 0)

        def half(g, b):
            c = 2 * g + b
            nb = b ^ 1

            @pl.when(c >= 1)
            def _():
                wait_store(c - 1, nb)

            @pl.when(jnp.logical_and(c + 1 < S, (c + 1) % ITILE == 0))
            def _():
                pltpu.make_async_copy(idx_src((c + 1) // ITILE),
                                      idx_v.at[((c + 1) // ITILE) % 2],
                                      isem).wait()

            wait_gather(c, b)

            @pl.when(c + 1 < S)
            def _():
                gather(c + 1, nb)

            @pl.when(jnp.logical_and((c + 1) % ITILE == 0,
                                     (c + 1) // ITILE + 1 < ntile))
            def _():
                pltpu.async_copy(idx_src((c + 1) // ITILE + 1),
                                 idx_v.at[(c // ITILE) % 2], isem)

            pr = [pe_v[c, pl.ds(16 * j, 16)] for j in range(D // 16)]

            def add_row(r, carry):
                for u in range(2):
                    for j in range(D // 16):
                        row = 2 * r + u
                        col = pl.ds(16 * j, 16)
                        rows_v[b, row, 0, col] = rows_v[b, row, 0, col] + pr[j]
                return carry

            lax.fori_loop(0, BLK // 2, add_row, 0)
            store(c, b)

        def outer(g, carry):
            half(g, 0)
            half(g, 1)
            return carry

        lax.fori_loop(0, S // 2, outer, 0)
        wait_store(S - 1, 1)

    return body(ids_t, pe, table3)


def kernel(input, tok_table):
    B, S = input.shape
    V, D = tok_table.shape
    pe = jnp.asarray(_sinusoidal_pe(_MAX_LEN, D)[:S])
    ids_t = input.T.astype(jnp.int32)
    table3 = tok_table.reshape(V, 1, D)
    return _run(ids_t, pe, table3, B, S)
